# baseline (speedup 1.0000x reference)
import functools

import jax
import jax.numpy as jnp
from jax import lax
from jax.experimental import pallas as pl
from jax.experimental.pallas import tpu as pltpu
from jax.experimental.pallas import tpu_sc as plsc

_LANES = 16


def _gather_sums(ids, cids, W_dae, Wc):
    B, L = ids.shape
    _, Lc = cids.shape
    N, E = W_dae.shape
    info = plsc.get_sparse_core_info()
    NC, NS = info.num_cores, info.num_subcores
    NW = NC * NS
    RB = B // NW

    mesh = plsc.VectorSubcoreMesh(core_axis_name="c", subcore_axis_name="s")

    @functools.partial(
        pl.kernel,
        out_type=[
            jax.ShapeDtypeStruct((B, E), jnp.float32),
            jax.ShapeDtypeStruct((B, E), jnp.float32),
        ],
        mesh=mesh,
        compiler_params=pltpu.CompilerParams(use_tc_tiling_on_sc=False),
        scratch_types=[
            pltpu.VMEM((RB, L), jnp.int32),
            pltpu.VMEM((RB, Lc), jnp.int32),
            pltpu.VMEM((RB, L, E), jnp.float32),
            pltpu.VMEM((RB, Lc, E), jnp.float32),
            pltpu.VMEM((RB, E), jnp.float32),
            pltpu.VMEM((RB, E), jnp.float32),
            pltpu.SemaphoreType.DMA,
            pltpu.SemaphoreType.DMA,
        ],
    )
    def k(ids_hbm, cids_hbm, wdae_hbm, wc_hbm, out_i, out_c,
          idx_i, idx_c, rows_i, rows_c, acc_i, acc_c, sem_i, sem_c):
        wid = lax.axis_index("s") * NC + lax.axis_index("c")
        base = wid * RB
        pltpu.sync_copy(ids_hbm.at[pl.ds(base, RB)], idx_i)
        pltpu.sync_copy(cids_hbm.at[pl.ds(base, RB)], idx_c)
        cps = []
        for b in range(RB):
            cps.append(pltpu.async_copy(wdae_hbm.at[idx_i.at[b]], rows_i.at[b], sem_i))
            cps.append(pltpu.async_copy(wc_hbm.at[idx_c.at[b]], rows_c.at[b], sem_c))
        for cp in cps:
            cp.wait()

        nh = E // _LANES

        def body(b, carry):
            for h in range(nh):
                sl = pl.ds(h * _LANES, _LANES)
                a = jnp.zeros((_LANES,), jnp.float32)
                for j in range(L):
                    a = a + rows_i[b, j, sl]
                acc_i[b, sl] = a
                a = jnp.zeros((_LANES,), jnp.float32)
                for j in range(Lc):
                    a = a + rows_c[b, j, sl]
                acc_c[b, sl] = a
            return carry

        lax.fori_loop(0, RB, body, None)
        pltpu.sync_copy(acc_i, out_i.at[pl.ds(base, RB)])
        pltpu.sync_copy(acc_c, out_c.at[pl.ds(base, RB)])

    return k(ids, cids, W_dae, Wc)


def _dae_proj(W_dae, Wd1):
    N, E = W_dae.shape
    D = Wd1.shape[1]
    RT = 4
    R = N // RT

    def body(w_ref, wd_ref, out_ref):
        i = pl.program_id(0)

        @pl.when(i == 0)
        def _():
            out_ref[...] = jnp.zeros_like(out_ref)

        out_ref[...] += lax.dot_general(
            w_ref[...], wd_ref[...], (((0,), (0,)), ((), ())),
            preferred_element_type=jnp.float32)

    return pl.pallas_call(
        body,
        grid=(RT,),
        in_specs=[
            pl.BlockSpec((R, E), lambda i: (i, 0)),
            pl.BlockSpec((R, D), lambda i: (i, 0)),
        ],
        out_specs=pl.BlockSpec((E, D), lambda i: (0, 0)),
        out_shape=jax.ShapeDtypeStruct((E, D), jnp.float32),
    )(W_dae, Wd1)


_COLS = 2048
_COLS_S = 4096


def _h_and_sums(s_dae, s_cnn, M, Wc1, bd1, bc1, Wf_b16, bf_pad, n_pad):
    B, E = s_dae.shape
    H, NP = Wf_b16.shape
    NT = NP // _COLS_S

    def body(sd_ref, sc_ref, m_ref, wc1_ref, bd1_ref, bc1_ref, wf_ref, bf_ref,
             h_ref, s_ref):
        j = pl.program_id(0)

        @pl.when(j == 0)
        def _():
            x = jnp.maximum(sd_ref[...], 0.0)
            y_dae = jnp.maximum(
                jnp.dot(x, m_ref[...], preferred_element_type=jnp.float32)
                + bd1_ref[...], 0.0)
            t = jnp.maximum(
                jnp.dot(sc_ref[...], wc1_ref[...],
                        preferred_element_type=jnp.float32) + bc1_ref[...], 0.0)
            t = t - jnp.max(t, axis=1, keepdims=True)
            e = jnp.exp(t)
            y_cnn = e / jnp.sum(e, axis=1, keepdims=True)
            h_ref[...] = jnp.concatenate([y_dae, y_cnn], axis=1)
            s_ref[...] = jnp.zeros_like(s_ref)

        y = jnp.maximum(
            jnp.dot(h_ref[...].astype(jnp.bfloat16), wf_ref[...],
                    preferred_element_type=jnp.float32) + bf_ref[...], 0.0)
        s_ref[...] += jnp.sum(jnp.exp(y), axis=1, keepdims=True)

        @pl.when(j == NT - 1)
        def _():
            s_ref[...] = 1.0 / (s_ref[...] - jnp.float32(n_pad))

    return pl.pallas_call(
        body,
        grid=(NT,),
        in_specs=[
            pl.BlockSpec((B, E), lambda j: (0, 0)),
            pl.BlockSpec((B, E), lambda j: (0, 0)),
            pl.BlockSpec(M.shape, lambda j: (0, 0)),
            pl.BlockSpec(Wc1.shape, lambda j: (0, 0)),
            pl.BlockSpec((1, E), lambda j: (0, 0)),
            pl.BlockSpec((1, E), lambda j: (0, 0)),
            pl.BlockSpec((H, _COLS_S), lambda j: (0, j)),
            pl.BlockSpec((1, _COLS_S), lambda j: (0, j)),
        ],
        out_specs=[
            pl.BlockSpec((B, 2 * E), lambda j: (0, 0)),
            pl.BlockSpec((B, 1), lambda j: (0, 0)),
        ],
        out_shape=[
            jax.ShapeDtypeStruct((B, 2 * E), jnp.float32),
            jax.ShapeDtypeStruct((B, 1), jnp.float32),
        ],
    )(s_dae, s_cnn, M, Wc1, bd1, bc1, Wf_b16, bf_pad)


def _head_out(h, sinv, Wf, bf2, N):
    B, H = h.shape
    NT = pl.cdiv(N, _COLS)

    def body(h_ref, s_ref, wf_ref, bf_ref, o_ref):
        y = jnp.maximum(
            jnp.dot(h_ref[...], wf_ref[...], preferred_element_type=jnp.float32)
            + bf_ref[...], 0.0)
        o_ref[...] = jnp.exp(y) * s_ref[...]

    return pl.pallas_call(
        body,
        grid=(NT,),
        in_specs=[
            pl.BlockSpec((B, H), lambda j: (0, 0)),
            pl.BlockSpec((B, 1), lambda j: (0, 0)),
            pl.BlockSpec((H, _COLS), lambda j: (0, j)),
            pl.BlockSpec((1, _COLS), lambda j: (0, j)),
        ],
        out_specs=pl.BlockSpec((B, _COLS), lambda j: (0, j)),
        out_shape=jax.ShapeDtypeStruct((B, N), jnp.float32),
    )(h, sinv, Wf, bf2)


def _fused_tc(s_dae, s_cnn, W_dae, Wd1, Wc1, bd1, bc1,
              Wf_b16, bf_pad, Wf, bf2, n_pad, N):
    B, E = s_dae.shape
    H = Wf.shape[0]
    RT = 10
    R = W_dae.shape[0] // RT
    NTS = Wf_b16.shape[1] // _COLS_S
    NTH = pl.cdiv(N, _COLS)
    P1 = RT
    P2 = RT + 1
    P3 = RT + 1 + NTS

    def body(wdae_ref, wd1_ref, sd_ref, sc_ref, wc1_ref, bd1_ref, bc1_ref,
             wfb_ref, bfp_ref, wf_ref, bff_ref, o_ref, m_sc, h_sc, s_sc,
             sw_sc):
        j = pl.program_id(0)

        @pl.when(j == 0)
        def _():
            m_sc[...] = jnp.zeros_like(m_sc)

        @pl.when(j < P1)
        def _():
            m_sc[...] += lax.dot_general(
                wdae_ref[...], wd1_ref[...], (((0,), (0,)), ((), ())),
                preferred_element_type=jnp.float32)

        @pl.when(j == P1)
        def _():
            x = jnp.maximum(sd_ref[...], 0.0)
            y_dae = jnp.maximum(
                jnp.dot(x, m_sc[...], preferred_element_type=jnp.float32)
                + bd1_ref[...], 0.0)
            t = jnp.maximum(
                jnp.dot(sc_ref[...], wc1_ref[...],
                        preferred_element_type=jnp.float32) + bc1_ref[...], 0.0)
            t = t - jnp.max(t, axis=1, keepdims=True)
            e = jnp.exp(t)
            y_cnn = e / jnp.sum(e, axis=1, keepdims=True)
            h_sc[...] = jnp.concatenate([y_dae, y_cnn], axis=1)
            sw_sc[...] = jnp.zeros_like(sw_sc)

        @pl.when(jnp.logical_and(j >= P2, j < P3))
        def _():
            y = jnp.maximum(
                jnp.dot(h_sc[...].astype(jnp.bfloat16), wfb_ref[...],
                        preferred_element_type=jnp.float32) + bfp_ref[...],
                0.0)
            p = jnp.exp(y)
            sw_sc[...] += jnp.sum(p.reshape(B, _COLS_S // 128, 128), axis=1)

            @pl.when(j == P3 - 1)
            def _():
                s = jnp.sum(sw_sc[...], axis=1, keepdims=True)
                s_sc[...] = 1.0 / (s - jnp.float32(n_pad))

        @pl.when(j >= P3)
        def _():
            y = jnp.maximum(
                jnp.dot(h_sc[...], wf_ref[...],
                        preferred_element_type=jnp.float32) + bff_ref[...],
                0.0)
            o_ref[...] = jnp.exp(y) * s_sc[...]

    def _clip(lo, hi, off):
        return lambda j: (0, jnp.clip(j - off, lo, hi))

    return pl.pallas_call(
        body,
        grid=(P3 + NTH,),
        in_specs=[
            pl.BlockSpec((R, E), lambda j: (jnp.clip(j, 0, RT - 1), 0)),
            pl.BlockSpec((R, Wd1.shape[1]), lambda j: (jnp.clip(j, 0, RT - 1), 0)),
            pl.BlockSpec((B, E), lambda j: (0, 0)),
            pl.BlockSpec((B, E), lambda j: (0, 0)),
            pl.BlockSpec(Wc1.shape, lambda j: (0, 0)),
            pl.BlockSpec((1, E), lambda j: (0, 0)),
            pl.BlockSpec((1, E), lambda j: (0, 0)),
            pl.BlockSpec((H, _COLS_S), _clip(0, NTS - 1, P2)),
            pl.BlockSpec((1, _COLS_S), _clip(0, NTS - 1, P2)),
            pl.BlockSpec((H, _COLS), _clip(0, NTH - 1, P3)),
            pl.BlockSpec((1, _COLS), _clip(0, NTH - 1, P3)),
        ],
        out_specs=pl.BlockSpec((B, _COLS), _clip(0, NTH - 1, P3)),
        out_shape=jax.ShapeDtypeStruct((B, N), jnp.float32),
        scratch_shapes=[
            pltpu.VMEM((E, Wd1.shape[1]), jnp.float32),
            pltpu.VMEM((B, 2 * E), jnp.float32),
            pltpu.VMEM((B, 1), jnp.float32),
            pltpu.VMEM((B, 128), jnp.float32),
        ],
    )(W_dae, Wd1, s_dae, s_cnn, Wc1, bd1, bc1, Wf_b16, bf_pad, Wf, bf2)


def kernel(ids, cids, W_dae, Wd1, bd1, Wc, Wc1, bc1, Wf, bf):
    ids = ids.astype(jnp.int32)
    cids = cids.astype(jnp.int32)
    N = Wf.shape[1]
    NP = ((N + _COLS_S - 1) // _COLS_S) * _COLS_S
    s_dae, s_cnn = _gather_sums(ids, cids, W_dae, Wc)
    Wf_b16 = jnp.pad(Wf, ((0, 0), (0, NP - N))).astype(jnp.bfloat16)
    bf_pad = jnp.pad(bf.reshape(1, -1), ((0, 0), (0, NP - N)),
                     constant_values=-1.0)
    return _fused_tc(s_dae, s_cnn, W_dae, Wd1, Wc1,
                     bd1.reshape(1, -1), bc1.reshape(1, -1),
                     Wf_b16, bf_pad, Wf, bf.reshape(1, -1), NP - N, N)


# speedup vs baseline: 1.1228x; 1.1228x over previous
import functools

import jax
import jax.numpy as jnp
from jax import lax
from jax.experimental import pallas as pl
from jax.experimental.pallas import tpu as pltpu
from jax.experimental.pallas import tpu_sc as plsc

_LANES = 16


def _gather_sums(ids, cids, W_dae, Wc):
    B, L = ids.shape
    _, Lc = cids.shape
    N, E = W_dae.shape
    info = plsc.get_sparse_core_info()
    NC, NS = info.num_cores, info.num_subcores
    NW = NC * NS
    RB = B // NW

    mesh = plsc.VectorSubcoreMesh(core_axis_name="c", subcore_axis_name="s")

    @functools.partial(
        pl.kernel,
        out_type=[
            jax.ShapeDtypeStruct((B, E), jnp.float32),
            jax.ShapeDtypeStruct((B, E), jnp.float32),
        ],
        mesh=mesh,
        compiler_params=pltpu.CompilerParams(use_tc_tiling_on_sc=False),
        scratch_types=[
            pltpu.VMEM((RB, L), jnp.int32),
            pltpu.VMEM((RB, Lc), jnp.int32),
            pltpu.VMEM((RB, L, E), jnp.float32),
            pltpu.VMEM((RB, Lc, E), jnp.float32),
            pltpu.VMEM((RB, E), jnp.float32),
            pltpu.VMEM((RB, E), jnp.float32),
            pltpu.SemaphoreType.DMA,
            pltpu.SemaphoreType.DMA,
        ],
    )
    def k(ids_hbm, cids_hbm, wdae_hbm, wc_hbm, out_i, out_c,
          idx_i, idx_c, rows_i, rows_c, acc_i, acc_c, sem_i, sem_c):
        wid = lax.axis_index("s") * NC + lax.axis_index("c")
        base = wid * RB
        pltpu.sync_copy(ids_hbm.at[pl.ds(base, RB)], idx_i)
        pltpu.sync_copy(cids_hbm.at[pl.ds(base, RB)], idx_c)
        cps = []
        for b in range(RB):
            cps.append(pltpu.async_copy(wdae_hbm.at[idx_i.at[b]], rows_i.at[b], sem_i))
            cps.append(pltpu.async_copy(wc_hbm.at[idx_c.at[b]], rows_c.at[b], sem_c))
        for cp in cps:
            cp.wait()

        nh = E // _LANES

        def body(b, carry):
            for h in range(nh):
                sl = pl.ds(h * _LANES, _LANES)
                a = jnp.zeros((_LANES,), jnp.float32)
                for j in range(L):
                    a = a + rows_i[b, j, sl]
                acc_i[b, sl] = a
                a = jnp.zeros((_LANES,), jnp.float32)
                for j in range(Lc):
                    a = a + rows_c[b, j, sl]
                acc_c[b, sl] = a
            return carry

        lax.fori_loop(0, RB, body, None)
        pltpu.sync_copy(acc_i, out_i.at[pl.ds(base, RB)])
        pltpu.sync_copy(acc_c, out_c.at[pl.ds(base, RB)])

    return k(ids, cids, W_dae, Wc)


def _dae_proj(W_dae, Wd1):
    N, E = W_dae.shape
    D = Wd1.shape[1]
    RT = 4
    R = N // RT

    def body(w_ref, wd_ref, out_ref):
        i = pl.program_id(0)

        @pl.when(i == 0)
        def _():
            out_ref[...] = jnp.zeros_like(out_ref)

        out_ref[...] += lax.dot_general(
            w_ref[...], wd_ref[...], (((0,), (0,)), ((), ())),
            preferred_element_type=jnp.float32)

    return pl.pallas_call(
        body,
        grid=(RT,),
        in_specs=[
            pl.BlockSpec((R, E), lambda i: (i, 0)),
            pl.BlockSpec((R, D), lambda i: (i, 0)),
        ],
        out_specs=pl.BlockSpec((E, D), lambda i: (0, 0)),
        out_shape=jax.ShapeDtypeStruct((E, D), jnp.float32),
    )(W_dae, Wd1)


_COLS = 2048
_COLS_S = 4096


def _h_and_sums(s_dae, s_cnn, M, Wc1, bd1, bc1, Wf_b16, bf_pad, n_pad):
    B, E = s_dae.shape
    H, NP = Wf_b16.shape
    NT = NP // _COLS_S

    def body(sd_ref, sc_ref, m_ref, wc1_ref, bd1_ref, bc1_ref, wf_ref, bf_ref,
             h_ref, s_ref):
        j = pl.program_id(0)

        @pl.when(j == 0)
        def _():
            x = jnp.maximum(sd_ref[...], 0.0)
            y_dae = jnp.maximum(
                jnp.dot(x, m_ref[...], preferred_element_type=jnp.float32)
                + bd1_ref[...], 0.0)
            t = jnp.maximum(
                jnp.dot(sc_ref[...], wc1_ref[...],
                        preferred_element_type=jnp.float32) + bc1_ref[...], 0.0)
            t = t - jnp.max(t, axis=1, keepdims=True)
            e = jnp.exp(t)
            y_cnn = e / jnp.sum(e, axis=1, keepdims=True)
            h_ref[...] = jnp.concatenate([y_dae, y_cnn], axis=1)
            s_ref[...] = jnp.zeros_like(s_ref)

        y = jnp.maximum(
            jnp.dot(h_ref[...].astype(jnp.bfloat16), wf_ref[...],
                    preferred_element_type=jnp.float32) + bf_ref[...], 0.0)
        s_ref[...] += jnp.sum(jnp.exp(y), axis=1, keepdims=True)

        @pl.when(j == NT - 1)
        def _():
            s_ref[...] = 1.0 / (s_ref[...] - jnp.float32(n_pad))

    return pl.pallas_call(
        body,
        grid=(NT,),
        in_specs=[
            pl.BlockSpec((B, E), lambda j: (0, 0)),
            pl.BlockSpec((B, E), lambda j: (0, 0)),
            pl.BlockSpec(M.shape, lambda j: (0, 0)),
            pl.BlockSpec(Wc1.shape, lambda j: (0, 0)),
            pl.BlockSpec((1, E), lambda j: (0, 0)),
            pl.BlockSpec((1, E), lambda j: (0, 0)),
            pl.BlockSpec((H, _COLS_S), lambda j: (0, j)),
            pl.BlockSpec((1, _COLS_S), lambda j: (0, j)),
        ],
        out_specs=[
            pl.BlockSpec((B, 2 * E), lambda j: (0, 0)),
            pl.BlockSpec((B, 1), lambda j: (0, 0)),
        ],
        out_shape=[
            jax.ShapeDtypeStruct((B, 2 * E), jnp.float32),
            jax.ShapeDtypeStruct((B, 1), jnp.float32),
        ],
    )(s_dae, s_cnn, M, Wc1, bd1, bc1, Wf_b16, bf_pad)


def _head_out(h, sinv, Wf, bf2, N):
    B, H = h.shape
    NT = pl.cdiv(N, _COLS)

    def body(h_ref, s_ref, wf_ref, bf_ref, o_ref):
        y = jnp.maximum(
            jnp.dot(h_ref[...], wf_ref[...], preferred_element_type=jnp.float32)
            + bf_ref[...], 0.0)
        o_ref[...] = jnp.exp(y) * s_ref[...]

    return pl.pallas_call(
        body,
        grid=(NT,),
        in_specs=[
            pl.BlockSpec((B, H), lambda j: (0, 0)),
            pl.BlockSpec((B, 1), lambda j: (0, 0)),
            pl.BlockSpec((H, _COLS), lambda j: (0, j)),
            pl.BlockSpec((1, _COLS), lambda j: (0, j)),
        ],
        out_specs=pl.BlockSpec((B, _COLS), lambda j: (0, j)),
        out_shape=jax.ShapeDtypeStruct((B, N), jnp.float32),
    )(h, sinv, Wf, bf2)


def _fused_tc(s_dae, s_cnn, W_dae, Wd1, Wc1, bd1, bc1,
              Wf_b16, bf_pad, Wf, bf2, n_pad, N):
    B, E = s_dae.shape
    H = Wf.shape[0]
    RT = 10
    R = W_dae.shape[0] // RT
    NTS = Wf_b16.shape[1] // _COLS_S
    NTH = pl.cdiv(N, _COLS)
    P1 = RT
    P2 = RT + 1
    P3 = RT + 1 + NTS

    def body(wdae_ref, wd1_ref, sd_ref, sc_ref, wc1_ref, bd1_ref, bc1_ref,
             wfb_ref, bfp_ref, wf_ref, bff_ref, o_ref, m_sc, h_sc, s_sc):
        j = pl.program_id(0)

        @pl.when(j == 0)
        def _():
            m_sc[...] = jnp.zeros_like(m_sc)

        @pl.when(j < P1)
        def _():
            m_sc[...] += lax.dot_general(
                wdae_ref[...], wd1_ref[...], (((0,), (0,)), ((), ())),
                preferred_element_type=jnp.float32)

        @pl.when(j == P1)
        def _():
            x = jnp.maximum(sd_ref[...], 0.0)
            y_dae = jnp.maximum(
                jnp.dot(x, m_sc[...], preferred_element_type=jnp.float32)
                + bd1_ref[...], 0.0)
            t = jnp.maximum(
                jnp.dot(sc_ref[...], wc1_ref[...],
                        preferred_element_type=jnp.float32) + bc1_ref[...], 0.0)
            t = t - jnp.max(t, axis=1, keepdims=True)
            e = jnp.exp(t)
            y_cnn = e / jnp.sum(e, axis=1, keepdims=True)
            h_sc[...] = jnp.concatenate([y_dae, y_cnn], axis=1)
            s_sc[...] = jnp.zeros_like(s_sc)

        @pl.when(jnp.logical_and(j >= P2, j < P3))
        def _():
            y = jnp.maximum(
                jnp.dot(h_sc[...].astype(jnp.bfloat16), wfb_ref[...],
                        preferred_element_type=jnp.float32) + bfp_ref[...],
                0.0)
            s_sc[...] += jnp.sum(jnp.exp(y), axis=1, keepdims=True)

            @pl.when(j == P3 - 1)
            def _():
                s_sc[...] = 1.0 / (s_sc[...] - jnp.float32(n_pad))

        @pl.when(j >= P3)
        def _():
            y = jnp.maximum(
                jnp.dot(h_sc[...], wf_ref[...],
                        preferred_element_type=jnp.float32) + bff_ref[...],
                0.0)
            o_ref[...] = jnp.exp(y) * s_sc[...]

    def _clip(lo, hi, off):
        return lambda j: (0, jnp.clip(j - off, lo, hi))

    return pl.pallas_call(
        body,
        grid=(P3 + NTH,),
        in_specs=[
            pl.BlockSpec((R, E), lambda j: (jnp.clip(j, 0, RT - 1), 0)),
            pl.BlockSpec((R, Wd1.shape[1]), lambda j: (jnp.clip(j, 0, RT - 1), 0)),
            pl.BlockSpec((B, E), lambda j: (0, 0)),
            pl.BlockSpec((B, E), lambda j: (0, 0)),
            pl.BlockSpec(Wc1.shape, lambda j: (0, 0)),
            pl.BlockSpec((1, E), lambda j: (0, 0)),
            pl.BlockSpec((1, E), lambda j: (0, 0)),
            pl.BlockSpec((H, _COLS_S), _clip(0, NTS - 1, P2)),
            pl.BlockSpec((1, _COLS_S), _clip(0, NTS - 1, P2)),
            pl.BlockSpec((H, _COLS), _clip(0, NTH - 1, P3)),
            pl.BlockSpec((1, _COLS), _clip(0, NTH - 1, P3)),
        ],
        out_specs=pl.BlockSpec((B, _COLS), _clip(0, NTH - 1, P3)),
        out_shape=jax.ShapeDtypeStruct((B, N), jnp.float32),
        scratch_shapes=[
            pltpu.VMEM((E, Wd1.shape[1]), jnp.float32),
            pltpu.VMEM((B, 2 * E), jnp.float32),
            pltpu.VMEM((B, 1), jnp.float32),
        ],
    )(W_dae, Wd1, s_dae, s_cnn, Wc1, bd1, bc1, Wf_b16, bf_pad, Wf, bf2)


def kernel(ids, cids, W_dae, Wd1, bd1, Wc, Wc1, bc1, Wf, bf):
    ids = ids.astype(jnp.int32)
    cids = cids.astype(jnp.int32)
    N = Wf.shape[1]
    NP = ((N + _COLS_S - 1) // _COLS_S) * _COLS_S
    s_dae, s_cnn = _gather_sums(ids, cids, W_dae, Wc)
    Wf_b16 = jnp.pad(Wf, ((0, 0), (0, NP - N))).astype(jnp.bfloat16)
    bf_pad = jnp.pad(bf.reshape(1, -1), ((0, 0), (0, NP - N)),
                     constant_values=-1.0)
    return _fused_tc(s_dae, s_cnn, W_dae, Wd1, Wc1,
                     bd1.reshape(1, -1), bc1.reshape(1, -1),
                     Wf_b16, bf_pad, Wf, bf.reshape(1, -1), NP - N, N)


# speedup vs baseline: 1.1589x; 1.0322x over previous
import functools

import jax
import jax.numpy as jnp
from jax import lax
from jax.experimental import pallas as pl
from jax.experimental.pallas import tpu as pltpu
from jax.experimental.pallas import tpu_sc as plsc

_LANES = 16


def _gather_sums(ids, cids, W_dae, Wc):
    B, L = ids.shape
    _, Lc = cids.shape
    N, E = W_dae.shape
    info = plsc.get_sparse_core_info()
    NC, NS = info.num_cores, info.num_subcores
    NW = NC * NS
    RB = B // NW

    mesh = plsc.VectorSubcoreMesh(core_axis_name="c", subcore_axis_name="s")

    @functools.partial(
        pl.kernel,
        out_type=[
            jax.ShapeDtypeStruct((B, E), jnp.float32),
            jax.ShapeDtypeStruct((B, E), jnp.float32),
        ],
        mesh=mesh,
        compiler_params=pltpu.CompilerParams(use_tc_tiling_on_sc=False),
        scratch_types=[
            pltpu.VMEM((RB, L), jnp.int32),
            pltpu.VMEM((RB, Lc), jnp.int32),
            pltpu.VMEM((RB, L, E), jnp.float32),
            pltpu.VMEM((RB, Lc, E), jnp.float32),
            pltpu.VMEM((RB, E), jnp.float32),
            pltpu.VMEM((RB, E), jnp.float32),
            pltpu.SemaphoreType.DMA,
            pltpu.SemaphoreType.DMA,
        ],
    )
    def k(ids_hbm, cids_hbm, wdae_hbm, wc_hbm, out_i, out_c,
          idx_i, idx_c, rows_i, rows_c, acc_i, acc_c, sem_i, sem_c):
        wid = lax.axis_index("s") * NC + lax.axis_index("c")
        base = wid * RB
        pltpu.sync_copy(ids_hbm.at[pl.ds(base, RB)], idx_i)
        pltpu.sync_copy(cids_hbm.at[pl.ds(base, RB)], idx_c)
        cps = []
        for b in range(RB):
            cps.append(pltpu.async_copy(wdae_hbm.at[idx_i.at[b]], rows_i.at[b], sem_i))
            cps.append(pltpu.async_copy(wc_hbm.at[idx_c.at[b]], rows_c.at[b], sem_c))
        for cp in cps:
            cp.wait()

        nh = E // _LANES

        def body(b, carry):
            for h in range(nh):
                sl = pl.ds(h * _LANES, _LANES)
                a = jnp.zeros((_LANES,), jnp.float32)
                for j in range(L):
                    a = a + rows_i[b, j, sl]
                acc_i[b, sl] = a
                a = jnp.zeros((_LANES,), jnp.float32)
                for j in range(Lc):
                    a = a + rows_c[b, j, sl]
                acc_c[b, sl] = a
            return carry

        lax.fori_loop(0, RB, body, None)
        pltpu.sync_copy(acc_i, out_i.at[pl.ds(base, RB)])
        pltpu.sync_copy(acc_c, out_c.at[pl.ds(base, RB)])

    return k(ids, cids, W_dae, Wc)


def _dae_proj(W_dae, Wd1):
    N, E = W_dae.shape
    D = Wd1.shape[1]
    RT = 4
    R = N // RT

    def body(w_ref, wd_ref, out_ref):
        i = pl.program_id(0)

        @pl.when(i == 0)
        def _():
            out_ref[...] = jnp.zeros_like(out_ref)

        out_ref[...] += lax.dot_general(
            w_ref[...], wd_ref[...], (((0,), (0,)), ((), ())),
            preferred_element_type=jnp.float32)

    return pl.pallas_call(
        body,
        grid=(RT,),
        in_specs=[
            pl.BlockSpec((R, E), lambda i: (i, 0)),
            pl.BlockSpec((R, D), lambda i: (i, 0)),
        ],
        out_specs=pl.BlockSpec((E, D), lambda i: (0, 0)),
        out_shape=jax.ShapeDtypeStruct((E, D), jnp.float32),
    )(W_dae, Wd1)


_COLS = 2048
_COLS_S = 4096


def _fused_tc(s_dae, s_cnn, W_dae, Wd1, Wc1, bd1, bc1, Wf, bf2, N):
    B, E = s_dae.shape
    H = Wf.shape[0]
    RT = 10
    R = W_dae.shape[0] // RT
    NTH = pl.cdiv(N, _COLS)
    P1 = RT
    P2 = RT + 1
    P3 = RT + 1 + NTH

    def body(wdae_ref, wd1_ref, sd_ref, sc_ref, wc1_ref, bd1_ref, bc1_ref,
             wf_ref, bff_ref, o_ref, m_sc, h_sc, s_sc):
        j = pl.program_id(0)

        @pl.when(j == 0)
        def _():
            m_sc[...] = jnp.zeros_like(m_sc)

        @pl.when(j < P1)
        def _():
            m_sc[...] += lax.dot_general(
                wdae_ref[...], wd1_ref[...], (((0,), (0,)), ((), ())),
                preferred_element_type=jnp.float32)

        @pl.when(j == P1)
        def _():
            x = jnp.maximum(sd_ref[...], 0.0)
            y_dae = jnp.maximum(
                jnp.dot(x, m_sc[...], preferred_element_type=jnp.float32)
                + bd1_ref[...], 0.0)
            t = jnp.maximum(
                jnp.dot(sc_ref[...], wc1_ref[...],
                        preferred_element_type=jnp.float32) + bc1_ref[...], 0.0)
            t = t - jnp.max(t, axis=1, keepdims=True)
            e = jnp.exp(t)
            y_cnn = e / jnp.sum(e, axis=1, keepdims=True)
            h_sc[...] = jnp.concatenate([y_dae, y_cnn], axis=1)
            s_sc[...] = jnp.zeros_like(s_sc)

        @pl.when(jnp.logical_and(j >= P2, j < P3))
        def _():
            y = jnp.maximum(
                jnp.dot(h_sc[...].astype(jnp.bfloat16),
                        wf_ref[...].astype(jnp.bfloat16),
                        preferred_element_type=jnp.float32) + bff_ref[...],
                0.0)
            p = jnp.exp(y)

            @pl.when(j < P3 - 1)
            def _():
                s_sc[...] += jnp.sum(p, axis=1, keepdims=True)

            @pl.when(j == P3 - 1)
            def _():
                col = ((NTH - 1) * _COLS
                       + lax.broadcasted_iota(jnp.int32, (B, _COLS), 1))
                s_sc[...] += jnp.sum(jnp.where(col < N, p, 0.0),
                                     axis=1, keepdims=True)
                s_sc[...] = 1.0 / s_sc[...]

        @pl.when(j >= P3)
        def _():
            y = jnp.maximum(
                jnp.dot(h_sc[...], wf_ref[...],
                        preferred_element_type=jnp.float32) + bff_ref[...],
                0.0)
            o_ref[...] = jnp.exp(y) * s_sc[...]

    def _sweep(j):
        k = jnp.where(j < P3, j - P2, j - P3)
        return (0, jnp.clip(k, 0, NTH - 1))

    def _clip(lo, hi, off):
        return lambda j: (0, jnp.clip(j - off, lo, hi))

    return pl.pallas_call(
        body,
        grid=(P3 + NTH,),
        in_specs=[
            pl.BlockSpec((R, E), lambda j: (jnp.clip(j, 0, RT - 1), 0)),
            pl.BlockSpec((R, Wd1.shape[1]), lambda j: (jnp.clip(j, 0, RT - 1), 0)),
            pl.BlockSpec((B, E), lambda j: (0, 0)),
            pl.BlockSpec((B, E), lambda j: (0, 0)),
            pl.BlockSpec(Wc1.shape, lambda j: (0, 0)),
            pl.BlockSpec((1, E), lambda j: (0, 0)),
            pl.BlockSpec((1, E), lambda j: (0, 0)),
            pl.BlockSpec((H, _COLS), _sweep),
            pl.BlockSpec((1, _COLS), _sweep),
        ],
        out_specs=pl.BlockSpec((B, _COLS), _clip(0, NTH - 1, P3)),
        out_shape=jax.ShapeDtypeStruct((B, N), jnp.float32),
        scratch_shapes=[
            pltpu.VMEM((E, Wd1.shape[1]), jnp.float32),
            pltpu.VMEM((B, 2 * E), jnp.float32),
            pltpu.VMEM((B, 1), jnp.float32),
        ],
    )(W_dae, Wd1, s_dae, s_cnn, Wc1, bd1, bc1, Wf, bf2)


def kernel(ids, cids, W_dae, Wd1, bd1, Wc, Wc1, bc1, Wf, bf):
    ids = ids.astype(jnp.int32)
    cids = cids.astype(jnp.int32)
    N = Wf.shape[1]
    s_dae, s_cnn = _gather_sums(ids, cids, W_dae, Wc)
    return _fused_tc(s_dae, s_cnn, W_dae, Wd1, Wc1,
                     bd1.reshape(1, -1), bc1.reshape(1, -1),
                     Wf, bf.reshape(1, -1), N)


# speedup vs baseline: 1.1600x; 1.0009x over previous
import functools

import jax
import jax.numpy as jnp
from jax import lax
from jax.experimental import pallas as pl
from jax.experimental.pallas import tpu as pltpu
from jax.experimental.pallas import tpu_sc as plsc

_LANES = 16


def _gather_sums(ids, cids, W_dae, Wc):
    B, L = ids.shape
    _, Lc = cids.shape
    N, E = W_dae.shape
    info = plsc.get_sparse_core_info()
    NC, NS = info.num_cores, info.num_subcores
    NW = NC * NS
    RB = B // NW

    mesh = plsc.VectorSubcoreMesh(core_axis_name="c", subcore_axis_name="s")

    @functools.partial(
        pl.kernel,
        out_type=[
            jax.ShapeDtypeStruct((B, E), jnp.float32),
            jax.ShapeDtypeStruct((B, E), jnp.float32),
        ],
        mesh=mesh,
        compiler_params=pltpu.CompilerParams(use_tc_tiling_on_sc=False),
        scratch_types=[
            pltpu.VMEM((RB, L), jnp.int32),
            pltpu.VMEM((RB, Lc), jnp.int32),
            pltpu.VMEM((RB, L, E), jnp.float32),
            pltpu.VMEM((RB, Lc, E), jnp.float32),
            pltpu.VMEM((RB, E), jnp.float32),
            pltpu.VMEM((RB, E), jnp.float32),
            pltpu.SemaphoreType.DMA,
            pltpu.SemaphoreType.DMA,
        ],
    )
    def k(ids_hbm, cids_hbm, wdae_hbm, wc_hbm, out_i, out_c,
          idx_i, idx_c, rows_i, rows_c, acc_i, acc_c, sem_i, sem_c):
        wid = lax.axis_index("s") * NC + lax.axis_index("c")
        base = wid * RB
        pltpu.sync_copy(ids_hbm.at[pl.ds(base, RB)], idx_i)
        pltpu.sync_copy(cids_hbm.at[pl.ds(base, RB)], idx_c)
        cps = []
        for b in range(RB):
            cps.append(pltpu.async_copy(wdae_hbm.at[idx_i.at[b]], rows_i.at[b], sem_i))
            cps.append(pltpu.async_copy(wc_hbm.at[idx_c.at[b]], rows_c.at[b], sem_c))
        for cp in cps:
            cp.wait()

        nh = E // _LANES

        def body(b, carry):
            for h in range(nh):
                sl = pl.ds(h * _LANES, _LANES)
                a = jnp.zeros((_LANES,), jnp.float32)
                for j in range(L):
                    a = a + rows_i[b, j, sl]
                acc_i[b, sl] = a
                a = jnp.zeros((_LANES,), jnp.float32)
                for j in range(Lc):
                    a = a + rows_c[b, j, sl]
                acc_c[b, sl] = a
            return carry

        lax.fori_loop(0, RB, body, None)
        pltpu.sync_copy(acc_i, out_i.at[pl.ds(base, RB)])
        pltpu.sync_copy(acc_c, out_c.at[pl.ds(base, RB)])

    return k(ids, cids, W_dae, Wc)


def _dae_proj(W_dae, Wd1):
    N, E = W_dae.shape
    D = Wd1.shape[1]
    RT = 4
    R = N // RT

    def body(w_ref, wd_ref, out_ref):
        i = pl.program_id(0)

        @pl.when(i == 0)
        def _():
            out_ref[...] = jnp.zeros_like(out_ref)

        out_ref[...] += lax.dot_general(
            w_ref[...], wd_ref[...], (((0,), (0,)), ((), ())),
            preferred_element_type=jnp.float32)

    return pl.pallas_call(
        body,
        grid=(RT,),
        in_specs=[
            pl.BlockSpec((R, E), lambda i: (i, 0)),
            pl.BlockSpec((R, D), lambda i: (i, 0)),
        ],
        out_specs=pl.BlockSpec((E, D), lambda i: (0, 0)),
        out_shape=jax.ShapeDtypeStruct((E, D), jnp.float32),
    )(W_dae, Wd1)


_COLS = 2048


def _fused_tc(s_dae, s_cnn, W_dae, Wd1, Wc1, bd1, bc1, Wf, bf2, N):
    B, E = s_dae.shape
    H = Wf.shape[0]
    RT = 10
    R = W_dae.shape[0] // RT
    NTH = pl.cdiv(N, _COLS)
    P1 = RT
    P2 = RT + 1
    P3 = RT + 1 + NTH

    def body(wdae_ref, wd1_ref, sd_ref, sc_ref, wc1_ref, bd1_ref, bc1_ref,
             wf_ref, bff_ref, o_ref, m_sc, h_sc, s_sc):
        j = pl.program_id(0)

        @pl.when(j == 0)
        def _():
            m_sc[...] = jnp.zeros_like(m_sc)

        @pl.when(j < P1)
        def _():
            m_sc[...] += lax.dot_general(
                wdae_ref[...], wd1_ref[...], (((0,), (0,)), ((), ())),
                preferred_element_type=jnp.float32)

        @pl.when(j == P1)
        def _():
            x = jnp.maximum(sd_ref[...], 0.0)
            y_dae = jnp.maximum(
                jnp.dot(x, m_sc[...], preferred_element_type=jnp.float32)
                + bd1_ref[...], 0.0)
            t = jnp.maximum(
                jnp.dot(sc_ref[...], wc1_ref[...],
                        preferred_element_type=jnp.float32) + bc1_ref[...], 0.0)
            t = t - jnp.max(t, axis=1, keepdims=True)
            e = jnp.exp(t)
            y_cnn = e / jnp.sum(e, axis=1, keepdims=True)
            h_sc[...] = jnp.concatenate([y_dae, y_cnn], axis=1)
            s_sc[...] = jnp.zeros_like(s_sc)

        @pl.when(jnp.logical_and(j >= P2, j < P3))
        def _():
            y = jnp.maximum(
                jnp.dot(h_sc[...].astype(jnp.bfloat16),
                        wf_ref[...].astype(jnp.bfloat16),
                        preferred_element_type=jnp.float32) + bff_ref[...],
                0.0)
            p = jnp.exp(y)

            @pl.when(j < P3 - 1)
            def _():
                s_sc[...] += jnp.sum(p, axis=1, keepdims=True)

            @pl.when(j == P3 - 1)
            def _():
                col = ((NTH - 1) * _COLS
                       + lax.broadcasted_iota(jnp.int32, (B, _COLS), 1))
                s_sc[...] += jnp.sum(jnp.where(col < N, p, 0.0),
                                     axis=1, keepdims=True)
                s_sc[...] = 1.0 / s_sc[...]

        @pl.when(j >= P3)
        def _():
            y = jnp.maximum(
                jnp.dot(h_sc[...], wf_ref[...],
                        preferred_element_type=jnp.float32) + bff_ref[...],
                0.0)
            o_ref[...] = jnp.exp(y) * s_sc[...]

    def _sweep(j):
        k = jnp.where(j < P3, j - P2, j - P3)
        return (0, jnp.clip(k, 0, NTH - 1))

    def _clip(lo, hi, off):
        return lambda j: (0, jnp.clip(j - off, lo, hi))

    return pl.pallas_call(
        body,
        grid=(P3 + NTH,),
        in_specs=[
            pl.BlockSpec((R, E), lambda j: (jnp.clip(j, 0, RT - 1), 0)),
            pl.BlockSpec((R, Wd1.shape[1]), lambda j: (jnp.clip(j, 0, RT - 1), 0)),
            pl.BlockSpec((B, E), lambda j: (0, 0)),
            pl.BlockSpec((B, E), lambda j: (0, 0)),
            pl.BlockSpec(Wc1.shape, lambda j: (0, 0)),
            pl.BlockSpec((1, E), lambda j: (0, 0)),
            pl.BlockSpec((1, E), lambda j: (0, 0)),
            pl.BlockSpec((H, _COLS), _sweep),
            pl.BlockSpec((1, _COLS), _sweep),
        ],
        out_specs=pl.BlockSpec((B, _COLS), _clip(0, NTH - 1, P3)),
        out_shape=jax.ShapeDtypeStruct((B, N), jnp.float32),
        scratch_shapes=[
            pltpu.VMEM((E, Wd1.shape[1]), jnp.float32),
            pltpu.VMEM((B, 2 * E), jnp.float32),
            pltpu.VMEM((B, 1), jnp.float32),
        ],
    )(W_dae, Wd1, s_dae, s_cnn, Wc1, bd1, bc1, Wf, bf2)


def kernel(ids, cids, W_dae, Wd1, bd1, Wc, Wc1, bc1, Wf, bf):
    ids = ids.astype(jnp.int32)
    cids = cids.astype(jnp.int32)
    N = Wf.shape[1]
    s_dae, s_cnn = _gather_sums(ids, cids, W_dae, Wc)
    return _fused_tc(s_dae, s_cnn, W_dae, Wd1, Wc1,
                     bd1.reshape(1, -1), bc1.reshape(1, -1),
                     Wf, bf.reshape(1, -1), N)
